# baseline (speedup 1.0000x reference)
import functools

import jax
import jax.numpy as jnp
from jax import lax
from jax.experimental import pallas as pl
from jax.experimental.pallas import tpu as pltpu
from jax.experimental.pallas import tpu_sc as plsc

NC = 2
NS = 16
LANES = 16
C = 128
D = 128
NDEN = NS * 640


def _tc_first_body(x_ref, w_ref, a_ref, h_ref, av_ref):
    h = jnp.dot(x_ref[...], w_ref[...], preferred_element_type=jnp.float32)
    h_ref[...] = h
    av_ref[...] = jnp.dot(h, a_ref[...], preferred_element_type=jnp.float32)


def _self_loop_norm(acc0_ref, acc1_ref, den0_ref, den1_ref, hp_ref, avp_ref,
                    b_ref):
    asum = avp_ref[..., 0:1] + avp_ref[..., 1:2]
    asum = jnp.where(asum < 0, asum * jnp.float32(0.2), asum)
    ps = jnp.exp(asum)
    den = den0_ref[...] + den1_ref[...] + ps + 1e-16
    g = (acc0_ref[...] + acc1_ref[...] + ps * hp_ref[...]) / den + b_ref[...]
    return jnp.where(g < 0, g * jnp.float32(0.01), g)


def _tc_mid_body(acc0_ref, acc1_ref, den0_ref, den1_ref, hp_ref, avp_ref,
                 b_ref, w_ref, a_ref, h_ref, av_ref):
    g = _self_loop_norm(acc0_ref, acc1_ref, den0_ref, den1_ref, hp_ref,
                        avp_ref, b_ref)
    h = jnp.dot(g, w_ref[...], preferred_element_type=jnp.float32)
    h_ref[...] = h
    av_ref[...] = jnp.dot(h, a_ref[...], preferred_element_type=jnp.float32)


def _tc_last_body(acc0_ref, acc1_ref, den0_ref, den1_ref, hp_ref, avp_ref,
                  b_ref, w_ref, bo_ref, o_ref):
    g = _self_loop_norm(acc0_ref, acc1_ref, den0_ref, den1_ref, hp_ref,
                        avp_ref, b_ref)
    o_ref[...] = jnp.dot(g, w_ref[...], preferred_element_type=jnp.float32) \
        + bo_ref[...]


def _row_spec(r):
    return pl.BlockSpec((r, D), lambda i: (i, 0))


def _den_spec(r):
    return pl.BlockSpec((r, 1), lambda i: (i, 0))


def _full_spec(shape):
    return pl.BlockSpec(shape, lambda i: tuple(0 for _ in shape))


def _tc_first(x, w, a, n, r):
    return pl.pallas_call(
        _tc_first_body,
        grid=(n // r,),
        in_specs=[_row_spec(r), _full_spec((D, D)), _full_spec((D, 2))],
        out_specs=[_row_spec(r), pl.BlockSpec((r, 2), lambda i: (i, 0))],
        out_shape=[jax.ShapeDtypeStruct((n, D), jnp.float32),
                   jax.ShapeDtypeStruct((n, 2), jnp.float32)],
    )(x, w, a)


def _av_spec(r):
    return pl.BlockSpec((r, 2), lambda i: (i, 0))


def _tc_mid(acc, den, hp, avp, b, w, a, n, r):
    return pl.pallas_call(
        _tc_mid_body,
        grid=(n // r,),
        in_specs=[_row_spec(r), _row_spec(r), _den_spec(r), _den_spec(r),
                  _row_spec(r), _av_spec(r),
                  _full_spec((1, D)), _full_spec((D, D)), _full_spec((D, 2))],
        out_specs=[_row_spec(r), _av_spec(r)],
        out_shape=[jax.ShapeDtypeStruct((n, D), jnp.float32),
                   jax.ShapeDtypeStruct((n, 2), jnp.float32)],
    )(acc[0], acc[1], den[0][:n].reshape(n, 1), den[1][:n].reshape(n, 1),
      hp, avp, b.reshape(1, D), w, a)


def _tc_last(acc, den, hp, avp, b, w_out, b_out, n, r):
    return pl.pallas_call(
        _tc_last_body,
        grid=(n // r,),
        in_specs=[_row_spec(r), _row_spec(r), _den_spec(r), _den_spec(r),
                  _row_spec(r), _av_spec(r),
                  _full_spec((1, D)), _full_spec((D, 1)), _full_spec((1, 1))],
        out_specs=[pl.BlockSpec((r, 1), lambda i: (i, 0))],
        out_shape=[jax.ShapeDtypeStruct((n, 1), jnp.float32)],
    )(acc[0], acc[1], den[0][:n].reshape(n, 1), den[1][:n].reshape(n, 1),
      hp, avp, b.reshape(1, D), w_out, b_out.reshape(1, 1))[0]


def _sc_edge_body(n, e_tot, k, h_hbm, asv_hbm, adv_hbm, src_hbm, dst_hbm,
                  acc_out, den_out,
                  src0, src1, src2, src3, dst0, dst1, dst2, dst3,
                  rin0, rin1, asb0, asb1, adb0, adb1, pv0, pv1, zbuf,
                  acc_sh, den_sh,
                  gs0, gs1, as0, as1, ad0, ad1, rs0, rs1, ds0, ds1,
                  is0, is1, is2, is3):
    srcv = (src0, src1, src2, src3)
    dstv = (dst0, dst1, dst2, dst3)
    rin = (rin0, rin1)
    asb = (asb0, asb1)
    adb = (adb0, adb1)
    pv = (pv0, pv1)
    gsem = (gs0, gs1)
    asem = (as0, as1)
    adsem = (ad0, ad1)
    rsem = (rs0, rs1)
    dsem = (ds0, ds1)
    isem = (is0, is1, is2, is3)

    c = lax.axis_index("c")
    s = lax.axis_index("s")
    pw = k * C
    ebase = (c * NS + s) * pw

    zero16 = jnp.zeros((LANES,), jnp.float32)

    def _zero_rows(r, _):
        for j in range(D // LANES):
            rin0[r, pl.ds(j * LANES, LANES)] = zero16
        return 0
    lax.fori_loop(0, C, _zero_rows, 0)
    for j in range(640 // LANES):
        zbuf[pl.ds(j * LANES, LANES)] = zero16

    dpt = (n // NS // 8) * 8
    head = n - NS * dpt
    t0 = head + s * dpt
    full = dpt // C
    rem = dpt - full * C
    for q in range(full):
        pltpu.sync_copy(rin0, acc_sh.at[pl.ds(t0 + q * C, C), :])
    if rem:
        pltpu.sync_copy(rin0.at[pl.ds(0, rem), :],
                        acc_sh.at[pl.ds(t0 + full * C, rem), :])
    pltpu.sync_copy(zbuf, den_sh.at[pl.ds(s * 640, 640)])

    @pl.when(s == 0)
    def _():
        pltpu.sync_copy(rin0.at[pl.ds(0, head), :],
                        acc_sh.at[pl.ds(0, head), :])

    plsc.subcore_barrier()

    iota16 = lax.iota(jnp.int32, LANES)

    def _idx_load(g, r):
        pltpu.make_async_copy(src_hbm.at[pl.ds(ebase + g * C, C)], srcv[r],
                              isem[r]).start()
        pltpu.make_async_copy(dst_hbm.at[pl.ds(ebase + g * C, C)], dstv[r],
                              isem[r]).start()

    def _idx_wait(g, r):
        pltpu.make_async_copy(src_hbm.at[pl.ds(ebase + g * C, C)], srcv[r],
                              isem[r]).wait()
        pltpu.make_async_copy(dst_hbm.at[pl.ds(ebase + g * C, C)], dstv[r],
                              isem[r]).wait()

    def _start_alpha(b, r):
        pltpu.make_async_copy(asv_hbm.at[srcv[r]], asb[b], asem[b]).start()
        pltpu.make_async_copy(adv_hbm.at[dstv[r]], adb[b], adsem[b]).start()

    def _start_rows(b, r):
        pltpu.make_async_copy(h_hbm.at[srcv[r]], rin[b], gsem[b]).start()

    def _wait_alpha(b, r):
        pltpu.make_async_copy(asv_hbm.at[srcv[r]], asb[b], asem[b]).wait()
        pltpu.make_async_copy(adv_hbm.at[dstv[r]], adb[b], adsem[b]).wait()

    def _wait_rows(b, r):
        pltpu.make_async_copy(h_hbm.at[srcv[r]], rin[b], gsem[b]).wait()

    def _scat_rows(b, r):
        return pltpu.make_async_copy(rin[b], acc_sh.at[dstv[r]], rsem[b])

    def _scat_den(b, r):
        return pltpu.make_async_copy(pv[b], den_sh.at[dstv[r]], dsem[b])

    _idx_load(0, 0)
    _idx_wait(0, 0)
    _start_alpha(0, 0)
    _start_rows(0, 0)
    _idx_load(1, 1)

    def _chunk(g, b, r):
        other = 1 - b

        @pl.when(g + 1 < k)
        def _():
            @pl.when(g >= 1)
            def _():
                _scat_rows(other, (r + 3) % 4).wait()
                _scat_den(other, (r + 3) % 4).wait()
            _idx_wait(g + 1, (r + 1) % 4)
            _start_alpha(other, (r + 1) % 4)
            _start_rows(other, (r + 1) % 4)

        @pl.when(g + 2 < k)
        def _():
            _idx_load(g + 2, (r + 2) % 4)

        _wait_alpha(b, r)

        for i in range(C // LANES):
            sl = pl.ds(i * LANES, LANES)
            e = asb[b][sl] + adb[b][sl]
            e = jnp.where(e < 0, e * jnp.float32(0.2), e)
            p = jnp.exp(e)
            gid = ebase + g * C + i * LANES + iota16
            p = jnp.where(gid < e_tot, p, jnp.float32(0.0))
            pv[b][sl] = p

        _wait_rows(b, r)

        @plsc.parallel_loop(0, C // LANES, unroll=4)
        def _scale(q):
            p16 = pv[b][pl.ds(q * LANES, LANES)]
            base = q * LANES
            for ri in range(LANES):
                pe = p16[ri]
                for j in range(D // LANES):
                    sl2 = pl.ds(j * LANES, LANES)
                    rin[b][base + ri, sl2] = rin[b][base + ri, sl2] * pe

        _scat_rows(b, r).start(add=True)
        _scat_den(b, r).start(add=True)

    def _round(m, _):
        for j in range(4):
            _chunk(4 * m + j, j % 2, j)
        return 0

    lax.fori_loop(0, k // 4, _round, 0)

    for g in (k - 2, k - 1):
        _scat_rows(g % 2, g % 4).wait()
        _scat_den(g % 2, g % 4).wait()

    plsc.subcore_barrier()

    for q in range(full):
        sl = pl.ds(t0 + q * C, C)
        pltpu.sync_copy(acc_sh.at[sl, :], acc_out.at[c].at[sl, :])
    if rem:
        sl = pl.ds(t0 + full * C, rem)
        pltpu.sync_copy(acc_sh.at[sl, :], acc_out.at[c].at[sl, :])
    pltpu.sync_copy(den_sh.at[pl.ds(s * 640, 640)],
                    den_out.at[c].at[pl.ds(s * 640, 640)])

    @pl.when(s == 0)
    def _():
        pltpu.sync_copy(acc_sh.at[pl.ds(0, head), :],
                        acc_out.at[c].at[pl.ds(0, head), :])


def _sc_edge(h, asv, adv, srcp, dstp, n, e_tot, k):
    mesh = plsc.VectorSubcoreMesh(core_axis_name="c", subcore_axis_name="s")
    f32 = jnp.float32
    i32 = jnp.int32
    kern = pl.kernel(
        functools.partial(_sc_edge_body, n, e_tot, k),
        out_type=[jax.ShapeDtypeStruct((NC, n, D), f32),
                  jax.ShapeDtypeStruct((NC, NDEN), f32)],
        mesh=mesh,
        compiler_params=pltpu.CompilerParams(
            needs_layout_passes=False,
            disable_bounds_checks=True,
        ),
        scratch_types=[
            pltpu.VMEM((C,), i32), pltpu.VMEM((C,), i32),
            pltpu.VMEM((C,), i32), pltpu.VMEM((C,), i32),
            pltpu.VMEM((C,), i32), pltpu.VMEM((C,), i32),
            pltpu.VMEM((C,), i32), pltpu.VMEM((C,), i32),
            pltpu.VMEM((C, D), f32), pltpu.VMEM((C, D), f32),
            pltpu.VMEM((C,), f32), pltpu.VMEM((C,), f32),
            pltpu.VMEM((C,), f32), pltpu.VMEM((C,), f32),
            pltpu.VMEM((C,), f32), pltpu.VMEM((C,), f32),
            pltpu.VMEM((640,), f32),
            pltpu.VMEM_SHARED((n, D), f32),
            pltpu.VMEM_SHARED((NDEN,), f32),
        ] + [pltpu.SemaphoreType.DMA] * 14,
    )
    return kern(h, asv, adv, srcp, dstp)


def kernel(x, edge_index, W0, att_src0, att_dst0, b0, W1, att_src1, att_dst1,
           b1, W2, att_src2, att_dst2, b2, W_out, b_out):
    n = x.shape[0]
    src = edge_index[0]
    dst = edge_index[1]
    e_tot = edge_index.shape[1]

    per_worker = -(-e_tot // (NC * NS * C))
    k = -(-per_worker // 4) * 4
    e_pad = NC * NS * C * k
    fill = jnp.arange(e_pad - e_tot, dtype=src.dtype) % n
    srcp = jnp.concatenate([src, fill])
    dstp = jnp.concatenate([dst, fill])

    r = 2000
    a0 = jnp.stack([att_src0, att_dst0], axis=1)
    a1 = jnp.stack([att_src1, att_dst1], axis=1)
    a2 = jnp.stack([att_src2, att_dst2], axis=1)

    h, av = _tc_first(x, W0, a0, n, r)
    acc, den = _sc_edge(h, av[:, 0], av[:, 1], srcp, dstp, n, e_tot, k)
    h2, av2 = _tc_mid(acc, den, h, av, b0, W1, a1, n, r)
    acc, den = _sc_edge(h2, av2[:, 0], av2[:, 1], srcp, dstp, n, e_tot, k)
    h3, av3 = _tc_mid(acc, den, h2, av2, b1, W2, a2, n, r)
    acc, den = _sc_edge(h3, av3[:, 0], av3[:, 1], srcp, dstp, n, e_tot, k)
    out = _tc_last(acc, den, h3, av3, b2, W_out, b_out, n, r)
    return out.reshape(n)


# speedup vs baseline: 1.1238x; 1.1238x over previous
import functools

import jax
import jax.numpy as jnp
from jax import lax
from jax.experimental import pallas as pl
from jax.experimental.pallas import tpu as pltpu
from jax.experimental.pallas import tpu_sc as plsc

NC = 2
NS = 16
LANES = 16
C = 128
D = 128
NDEN = NS * 640


def _tc_first_body(x_ref, w_ref, a_ref, h_ref, av_ref):
    h = jnp.dot(x_ref[...], w_ref[...], preferred_element_type=jnp.float32)
    h_ref[...] = h
    av_ref[...] = jnp.dot(h, a_ref[...], preferred_element_type=jnp.float32)


def _self_loop_norm(acc0_ref, acc1_ref, den0_ref, den1_ref, hp_ref, avp_ref,
                    b_ref):
    asum = avp_ref[..., 0:1] + avp_ref[..., 1:2]
    asum = jnp.where(asum < 0, asum * jnp.float32(0.2), asum)
    ps = jnp.exp(asum)
    den = den0_ref[...] + den1_ref[...] + ps + 1e-16
    g = (acc0_ref[...] + acc1_ref[...] + ps * hp_ref[...]) / den + b_ref[...]
    return jnp.where(g < 0, g * jnp.float32(0.01), g)


def _tc_mid_body(acc0_ref, acc1_ref, den0_ref, den1_ref, hp_ref, avp_ref,
                 b_ref, w_ref, a_ref, h_ref, av_ref):
    g = _self_loop_norm(acc0_ref, acc1_ref, den0_ref, den1_ref, hp_ref,
                        avp_ref, b_ref)
    h = jnp.dot(g, w_ref[...], preferred_element_type=jnp.float32)
    h_ref[...] = h
    av_ref[...] = jnp.dot(h, a_ref[...], preferred_element_type=jnp.float32)


def _tc_last_body(acc0_ref, acc1_ref, den0_ref, den1_ref, hp_ref, avp_ref,
                  b_ref, w_ref, bo_ref, o_ref):
    g = _self_loop_norm(acc0_ref, acc1_ref, den0_ref, den1_ref, hp_ref,
                        avp_ref, b_ref)
    o_ref[...] = jnp.dot(g, w_ref[...], preferred_element_type=jnp.float32) \
        + bo_ref[...]


def _row_spec(r):
    return pl.BlockSpec((r, D), lambda i: (i, 0))


def _den_spec(r):
    return pl.BlockSpec((r, 1), lambda i: (i, 0))


def _full_spec(shape):
    return pl.BlockSpec(shape, lambda i: tuple(0 for _ in shape))


def _tc_first(x, w, a, n, r):
    return pl.pallas_call(
        _tc_first_body,
        grid=(n // r,),
        in_specs=[_row_spec(r), _full_spec((D, D)), _full_spec((D, 2))],
        out_specs=[_row_spec(r), pl.BlockSpec((r, 2), lambda i: (i, 0))],
        out_shape=[jax.ShapeDtypeStruct((n, D), jnp.float32),
                   jax.ShapeDtypeStruct((n, 2), jnp.float32)],
    )(x, w, a)


def _av_spec(r):
    return pl.BlockSpec((r, 2), lambda i: (i, 0))


def _tc_mid(acc, den, hp, avp, b, w, a, n, r):
    return pl.pallas_call(
        _tc_mid_body,
        grid=(n // r,),
        in_specs=[_row_spec(r), _row_spec(r), _den_spec(r), _den_spec(r),
                  _row_spec(r), _av_spec(r),
                  _full_spec((1, D)), _full_spec((D, D)), _full_spec((D, 2))],
        out_specs=[_row_spec(r), _av_spec(r)],
        out_shape=[jax.ShapeDtypeStruct((n, D), jnp.float32),
                   jax.ShapeDtypeStruct((n, 2), jnp.float32)],
    )(acc[0], acc[1], den[0][:n].reshape(n, 1), den[1][:n].reshape(n, 1),
      hp, avp, b.reshape(1, D), w, a)


def _tc_last(acc, den, hp, avp, b, w_out, b_out, n, r):
    return pl.pallas_call(
        _tc_last_body,
        grid=(n // r,),
        in_specs=[_row_spec(r), _row_spec(r), _den_spec(r), _den_spec(r),
                  _row_spec(r), _av_spec(r),
                  _full_spec((1, D)), _full_spec((D, 1)), _full_spec((1, 1))],
        out_specs=[pl.BlockSpec((r, 1), lambda i: (i, 0))],
        out_shape=[jax.ShapeDtypeStruct((n, 1), jnp.float32)],
    )(acc[0], acc[1], den[0][:n].reshape(n, 1), den[1][:n].reshape(n, 1),
      hp, avp, b.reshape(1, D), w_out, b_out.reshape(1, 1))[0]


def _sc_edge_body(n, e_tot, k, h_hbm, asv_hbm, adv_hbm, src_hbm, dst_hbm,
                  acc_out, den_out,
                  src0, src1, src2, src3, dst0, dst1, dst2, dst3,
                  rin0, rin1, asb0, asb1, adb0, adb1, pv0, pv1, zbuf,
                  acc_sh, den_sh,
                  gs0, gs1, as0, as1, ad0, ad1, rs0, rs1, ds0, ds1,
                  is0, is1, is2, is3):
    srcv = (src0, src1, src2, src3)
    dstv = (dst0, dst1, dst2, dst3)
    rin = (rin0, rin1)
    asb = (asb0, asb1)
    adb = (adb0, adb1)
    pv = (pv0, pv1)
    gsem = (gs0, gs1)
    asem = (as0, as1)
    adsem = (ad0, ad1)
    rsem = (rs0, rs1)
    dsem = (ds0, ds1)
    isem = (is0, is1, is2, is3)

    c = lax.axis_index("c")
    s = lax.axis_index("s")
    pw = k * C
    ebase = (c * NS + s) * pw

    zero16 = jnp.zeros((LANES,), jnp.float32)

    def _zero_rows(r, _):
        for j in range(D // LANES):
            rin0[r, pl.ds(j * LANES, LANES)] = zero16
        return 0
    lax.fori_loop(0, C, _zero_rows, 0)
    for j in range(640 // LANES):
        zbuf[pl.ds(j * LANES, LANES)] = zero16

    dpt = (n // NS // 8) * 8
    head = n - NS * dpt
    t0 = head + s * dpt
    full = dpt // C
    rem = dpt - full * C
    for q in range(full):
        pltpu.sync_copy(rin0, acc_sh.at[pl.ds(t0 + q * C, C), :])
    if rem:
        pltpu.sync_copy(rin0.at[pl.ds(0, rem), :],
                        acc_sh.at[pl.ds(t0 + full * C, rem), :])
    pltpu.sync_copy(zbuf, den_sh.at[pl.ds(s * 640, 640)])

    @pl.when(s == 0)
    def _():
        pltpu.sync_copy(rin0.at[pl.ds(0, head), :],
                        acc_sh.at[pl.ds(0, head), :])

    plsc.subcore_barrier()

    iota16 = lax.iota(jnp.int32, LANES)

    def _idx_load(g, r):
        pltpu.make_async_copy(src_hbm.at[pl.ds(ebase + g * C, C)], srcv[r],
                              isem[r]).start()
        pltpu.make_async_copy(dst_hbm.at[pl.ds(ebase + g * C, C)], dstv[r],
                              isem[r]).start()

    def _idx_wait(g, r):
        pltpu.make_async_copy(src_hbm.at[pl.ds(ebase + g * C, C)], srcv[r],
                              isem[r]).wait()
        pltpu.make_async_copy(dst_hbm.at[pl.ds(ebase + g * C, C)], dstv[r],
                              isem[r]).wait()

    def _start_alpha(b, r):
        pltpu.make_async_copy(asv_hbm.at[srcv[r]], asb[b], asem[b]).start()
        pltpu.make_async_copy(adv_hbm.at[dstv[r]], adb[b], adsem[b]).start()

    def _start_rows(b, r):
        pltpu.make_async_copy(h_hbm.at[srcv[r]], rin[b], gsem[b]).start()

    def _wait_alpha(b, r):
        pltpu.make_async_copy(asv_hbm.at[srcv[r]], asb[b], asem[b]).wait()
        pltpu.make_async_copy(adv_hbm.at[dstv[r]], adb[b], adsem[b]).wait()

    def _wait_rows(b, r):
        pltpu.make_async_copy(h_hbm.at[srcv[r]], rin[b], gsem[b]).wait()

    def _scat_rows(b, r):
        return pltpu.make_async_copy(rin[b], acc_sh.at[dstv[r]], rsem[b])

    def _scat_den(b, r):
        return pltpu.make_async_copy(pv[b], den_sh.at[dstv[r]], dsem[b])

    _idx_load(0, 0)
    _idx_wait(0, 0)
    _start_alpha(0, 0)
    _start_rows(0, 0)
    _idx_load(1, 1)

    def _chunk(g, b, r):
        other = 1 - b

        @pl.when(g + 1 < k)
        def _():
            @pl.when(g >= 1)
            def _():
                _scat_rows(other, (r + 3) % 4).wait()
                _scat_den(other, (r + 3) % 4).wait()
            _idx_wait(g + 1, (r + 1) % 4)
            _start_alpha(other, (r + 1) % 4)
            _start_rows(other, (r + 1) % 4)

        @pl.when(g + 2 < k)
        def _():
            _idx_load(g + 2, (r + 2) % 4)

        _wait_alpha(b, r)

        for i in range(C // LANES):
            sl = pl.ds(i * LANES, LANES)
            e = asb[b][sl] + adb[b][sl]
            e = jnp.where(e < 0, e * jnp.float32(0.2), e)
            p = jnp.exp(e)
            gid = ebase + g * C + i * LANES + iota16
            p = jnp.where(gid < e_tot, p, jnp.float32(0.0))
            pv[b][sl] = p

        _wait_rows(b, r)

        @plsc.parallel_loop(0, C // LANES, unroll=2)
        def _scale(q):
            p16 = pv[b][pl.ds(q * LANES, LANES)]
            base = q * LANES
            for ri in range(LANES):
                pe = p16[ri]
                for j in range(D // LANES):
                    sl2 = pl.ds(j * LANES, LANES)
                    rin[b][base + ri, sl2] = rin[b][base + ri, sl2] * pe

        _scat_rows(b, r).start(add=True)
        _scat_den(b, r).start(add=True)

    def _round(m, _):
        for j in range(4):
            _chunk(4 * m + j, j % 2, j)
        return 0

    lax.fori_loop(0, k // 4, _round, 0)

    for g in (k - 2, k - 1):
        _scat_rows(g % 2, g % 4).wait()
        _scat_den(g % 2, g % 4).wait()

    plsc.subcore_barrier()

    for q in range(full):
        sl = pl.ds(t0 + q * C, C)
        pltpu.sync_copy(acc_sh.at[sl, :], acc_out.at[c].at[sl, :])
    if rem:
        sl = pl.ds(t0 + full * C, rem)
        pltpu.sync_copy(acc_sh.at[sl, :], acc_out.at[c].at[sl, :])
    pltpu.sync_copy(den_sh.at[pl.ds(s * 640, 640)],
                    den_out.at[c].at[pl.ds(s * 640, 640)])

    @pl.when(s == 0)
    def _():
        pltpu.sync_copy(acc_sh.at[pl.ds(0, head), :],
                        acc_out.at[c].at[pl.ds(0, head), :])


def _sc_edge(h, asv, adv, srcp, dstp, n, e_tot, k):
    mesh = plsc.VectorSubcoreMesh(core_axis_name="c", subcore_axis_name="s")
    f32 = jnp.float32
    i32 = jnp.int32
    kern = pl.kernel(
        functools.partial(_sc_edge_body, n, e_tot, k),
        out_type=[jax.ShapeDtypeStruct((NC, n, D), f32),
                  jax.ShapeDtypeStruct((NC, NDEN), f32)],
        mesh=mesh,
        compiler_params=pltpu.CompilerParams(
            needs_layout_passes=False,
            disable_bounds_checks=True,
        ),
        scratch_types=[
            pltpu.VMEM((C,), i32), pltpu.VMEM((C,), i32),
            pltpu.VMEM((C,), i32), pltpu.VMEM((C,), i32),
            pltpu.VMEM((C,), i32), pltpu.VMEM((C,), i32),
            pltpu.VMEM((C,), i32), pltpu.VMEM((C,), i32),
            pltpu.VMEM((C, D), f32), pltpu.VMEM((C, D), f32),
            pltpu.VMEM((C,), f32), pltpu.VMEM((C,), f32),
            pltpu.VMEM((C,), f32), pltpu.VMEM((C,), f32),
            pltpu.VMEM((C,), f32), pltpu.VMEM((C,), f32),
            pltpu.VMEM((640,), f32),
            pltpu.VMEM_SHARED((n, D), f32),
            pltpu.VMEM_SHARED((NDEN,), f32),
        ] + [pltpu.SemaphoreType.DMA] * 14,
    )
    return kern(h, asv, adv, srcp, dstp)


def kernel(x, edge_index, W0, att_src0, att_dst0, b0, W1, att_src1, att_dst1,
           b1, W2, att_src2, att_dst2, b2, W_out, b_out):
    n = x.shape[0]
    src = edge_index[0]
    dst = edge_index[1]
    e_tot = edge_index.shape[1]

    per_worker = -(-e_tot // (NC * NS * C))
    k = -(-per_worker // 4) * 4
    e_pad = NC * NS * C * k
    fill = jnp.arange(e_pad - e_tot, dtype=src.dtype) % n
    srcp = jnp.concatenate([src, fill])
    dstp = jnp.concatenate([dst, fill])

    r = 2000
    a0 = jnp.stack([att_src0, att_dst0], axis=1)
    a1 = jnp.stack([att_src1, att_dst1], axis=1)
    a2 = jnp.stack([att_src2, att_dst2], axis=1)

    h, av = _tc_first(x, W0, a0, n, r)
    acc, den = _sc_edge(h, av[:, 0], av[:, 1], srcp, dstp, n, e_tot, k)
    h2, av2 = _tc_mid(acc, den, h, av, b0, W1, a1, n, r)
    acc, den = _sc_edge(h2, av2[:, 0], av2[:, 1], srcp, dstp, n, e_tot, k)
    h3, av3 = _tc_mid(acc, den, h2, av2, b1, W2, a2, n, r)
    acc, den = _sc_edge(h3, av3[:, 0], av3[:, 1], srcp, dstp, n, e_tot, k)
    out = _tc_last(acc, den, h3, av3, b2, W_out, b_out, n, r)
    return out.reshape(n)


# speedup vs baseline: 1.1261x; 1.0021x over previous
import functools

import jax
import jax.numpy as jnp
from jax import lax
from jax.experimental import pallas as pl
from jax.experimental.pallas import tpu as pltpu
from jax.experimental.pallas import tpu_sc as plsc

NC = 2
NS = 16
LANES = 16
C = 112
D = 128
NDEN = NS * 640


def _tc_first_body(x_ref, w_ref, a_ref, h_ref, av_ref):
    h = jnp.dot(x_ref[...], w_ref[...], preferred_element_type=jnp.float32)
    h_ref[...] = h
    av_ref[...] = jnp.dot(h, a_ref[...], preferred_element_type=jnp.float32)


def _self_loop_norm(acc0_ref, acc1_ref, den0_ref, den1_ref, hp_ref, avp_ref,
                    b_ref):
    asum = avp_ref[..., 0:1] + avp_ref[..., 1:2]
    asum = jnp.where(asum < 0, asum * jnp.float32(0.2), asum)
    ps = jnp.exp(asum)
    den = den0_ref[...] + den1_ref[...] + ps + 1e-16
    g = (acc0_ref[...] + acc1_ref[...] + ps * hp_ref[...]) / den + b_ref[...]
    return jnp.where(g < 0, g * jnp.float32(0.01), g)


def _tc_mid_body(acc0_ref, acc1_ref, den0_ref, den1_ref, hp_ref, avp_ref,
                 b_ref, w_ref, a_ref, h_ref, av_ref):
    g = _self_loop_norm(acc0_ref, acc1_ref, den0_ref, den1_ref, hp_ref,
                        avp_ref, b_ref)
    h = jnp.dot(g, w_ref[...], preferred_element_type=jnp.float32)
    h_ref[...] = h
    av_ref[...] = jnp.dot(h, a_ref[...], preferred_element_type=jnp.float32)


def _tc_last_body(acc0_ref, acc1_ref, den0_ref, den1_ref, hp_ref, avp_ref,
                  b_ref, w_ref, bo_ref, o_ref):
    g = _self_loop_norm(acc0_ref, acc1_ref, den0_ref, den1_ref, hp_ref,
                        avp_ref, b_ref)
    o_ref[...] = jnp.dot(g, w_ref[...], preferred_element_type=jnp.float32) \
        + bo_ref[...]


def _row_spec(r):
    return pl.BlockSpec((r, D), lambda i: (i, 0))


def _den_spec(r):
    return pl.BlockSpec((r, 1), lambda i: (i, 0))


def _full_spec(shape):
    return pl.BlockSpec(shape, lambda i: tuple(0 for _ in shape))


def _tc_first(x, w, a, n, r):
    return pl.pallas_call(
        _tc_first_body,
        grid=(n // r,),
        in_specs=[_row_spec(r), _full_spec((D, D)), _full_spec((D, 2))],
        out_specs=[_row_spec(r), pl.BlockSpec((r, 2), lambda i: (i, 0))],
        out_shape=[jax.ShapeDtypeStruct((n, D), jnp.float32),
                   jax.ShapeDtypeStruct((n, 2), jnp.float32)],
    )(x, w, a)


def _av_spec(r):
    return pl.BlockSpec((r, 2), lambda i: (i, 0))


def _tc_mid(acc, den, hp, avp, b, w, a, n, r):
    return pl.pallas_call(
        _tc_mid_body,
        grid=(n // r,),
        in_specs=[_row_spec(r), _row_spec(r), _den_spec(r), _den_spec(r),
                  _row_spec(r), _av_spec(r),
                  _full_spec((1, D)), _full_spec((D, D)), _full_spec((D, 2))],
        out_specs=[_row_spec(r), _av_spec(r)],
        out_shape=[jax.ShapeDtypeStruct((n, D), jnp.float32),
                   jax.ShapeDtypeStruct((n, 2), jnp.float32)],
    )(acc[0], acc[1], den[0][:n].reshape(n, 1), den[1][:n].reshape(n, 1),
      hp, avp, b.reshape(1, D), w, a)


def _tc_last(acc, den, hp, avp, b, w_out, b_out, n, r):
    return pl.pallas_call(
        _tc_last_body,
        grid=(n // r,),
        in_specs=[_row_spec(r), _row_spec(r), _den_spec(r), _den_spec(r),
                  _row_spec(r), _av_spec(r),
                  _full_spec((1, D)), _full_spec((D, 1)), _full_spec((1, 1))],
        out_specs=[pl.BlockSpec((r, 1), lambda i: (i, 0))],
        out_shape=[jax.ShapeDtypeStruct((n, 1), jnp.float32)],
    )(acc[0], acc[1], den[0][:n].reshape(n, 1), den[1][:n].reshape(n, 1),
      hp, avp, b.reshape(1, D), w_out, b_out.reshape(1, 1))[0]


def _sc_edge_body(n, e_tot, k, h_hbm, asv_hbm, adv_hbm, eidx_hbm,
                  acc_out, den_out,
                  idx0, idx1, idx2, rin0, rin1, rin2,
                  asb0, asb1, adb0, adb1, pv0, pv1, dso0, dso1, zbuf,
                  acc_sh, den_sh,
                  is0, is1, is2, gs0, gs1, gs2,
                  as0, as1, ad0, ad1, rs0, rs1, ds0, ds1):
    idxv = (idx0, idx1, idx2)
    rin = (rin0, rin1, rin2)
    asb = (asb0, asb1)
    adb = (adb0, adb1)
    pv = (pv0, pv1)
    dso = (dso0, dso1)
    isem = (is0, is1, is2)
    gsem = (gs0, gs1, gs2)
    asem = (as0, as1)
    adsem = (ad0, ad1)
    rsem = (rs0, rs1)
    dsem = (ds0, ds1)

    c = lax.axis_index("c")
    s = lax.axis_index("s")
    pw = k * C
    ebase = (c * NS + s) * pw

    zero16 = jnp.zeros((LANES,), jnp.float32)

    def _zero_rows(r, _):
        for j in range(D // LANES):
            rin0[r, pl.ds(j * LANES, LANES)] = zero16
        return 0
    lax.fori_loop(0, C, _zero_rows, 0)
    for j in range(640 // LANES):
        zbuf[pl.ds(j * LANES, LANES)] = zero16

    dpt = (n // NS // 8) * 8
    head = n - NS * dpt
    t0 = head + s * dpt
    full = dpt // C
    rem = dpt - full * C
    for q in range(full):
        pltpu.sync_copy(rin0, acc_sh.at[pl.ds(t0 + q * C, C), :])
    if rem:
        pltpu.sync_copy(rin0.at[pl.ds(0, rem), :],
                        acc_sh.at[pl.ds(t0 + full * C, rem), :])
    pltpu.sync_copy(zbuf, den_sh.at[pl.ds(s * 640, 640)])

    @pl.when(s == 0)
    def _():
        pltpu.sync_copy(rin0.at[pl.ds(0, head), :],
                        acc_sh.at[pl.ds(0, head), :])

    plsc.subcore_barrier()

    iota16 = lax.iota(jnp.int32, LANES)
    chunk0 = (c * NS + s) * k

    def _idx(g, q):
        return pltpu.make_async_copy(eidx_hbm.at[chunk0 + g], idxv[q],
                                     isem[q])

    def _alpha(b, q):
        return (pltpu.make_async_copy(asv_hbm.at[idxv[q].at[0]], asb[b],
                                      asem[b]),
                pltpu.make_async_copy(adv_hbm.at[idxv[q].at[1]], adb[b],
                                     adsem[b]))

    def _rows(q):
        return pltpu.make_async_copy(h_hbm.at[idxv[q].at[0]], rin[q],
                                     gsem[q])

    def _scat_rows(b, q):
        return pltpu.make_async_copy(rin[q], acc_sh.at[dso[b]], rsem[b])

    def _scat_den(b):
        return pltpu.make_async_copy(pv[b], den_sh.at[dso[b]], dsem[b])

    _idx(0, 0).start()
    _idx(0, 0).wait()
    for d in _alpha(0, 0):
        d.start()
    _rows(0).start()
    _idx(1, 1).start()

    def _chunk(g, b, q):
        other = 1 - b
        qn = (q + 1) % 3

        @pl.when(g >= 2)
        def _():
            _scat_rows(b, qn).wait()
            _scat_den(b).wait()

        @pl.when(g + 1 < k)
        def _():
            _idx(g + 1, qn).wait()
            for d in _alpha(other, qn):
                d.start()
            _rows(qn).start()

        @pl.when(g + 2 < k)
        def _():
            _idx(g + 2, (q + 2) % 3).start()

        for d in _alpha(b, q):
            d.wait()

        for i in range(C // LANES):
            sl = pl.ds(i * LANES, LANES)
            e = asb[b][sl] + adb[b][sl]
            e = jnp.where(e < 0, e * jnp.float32(0.2), e)
            p = jnp.exp(e)
            gid = ebase + g * C + i * LANES + iota16
            p = jnp.where(gid < e_tot, p, jnp.float32(0.0))
            pv[b][sl] = p
            dso[b][sl] = idxv[q][1, sl]

        _rows(q).wait()

        @plsc.parallel_loop(0, C // LANES, unroll=2)
        def _scale(qq):
            p16 = pv[b][pl.ds(qq * LANES, LANES)]
            base = qq * LANES
            for ri in range(LANES):
                pe = p16[ri]
                for j in range(D // LANES):
                    sl2 = pl.ds(j * LANES, LANES)
                    rin[q][base + ri, sl2] = rin[q][base + ri, sl2] * pe

        _scat_rows(b, q).start(add=True)
        _scat_den(b).start(add=True)

    def _round(m, _):
        for j in range(6):
            _chunk(6 * m + j, j % 2, j % 3)
        return 0

    lax.fori_loop(0, k // 6, _round, 0)

    for g in (k - 2, k - 1):
        _scat_rows(g % 2, g % 3).wait()
        _scat_den(g % 2).wait()

    plsc.subcore_barrier()

    for q in range(full):
        sl = pl.ds(t0 + q * C, C)
        pltpu.sync_copy(acc_sh.at[sl, :], acc_out.at[c].at[sl, :])
    if rem:
        sl = pl.ds(t0 + full * C, rem)
        pltpu.sync_copy(acc_sh.at[sl, :], acc_out.at[c].at[sl, :])
    pltpu.sync_copy(den_sh.at[pl.ds(s * 640, 640)],
                    den_out.at[c].at[pl.ds(s * 640, 640)])

    @pl.when(s == 0)
    def _():
        pltpu.sync_copy(acc_sh.at[pl.ds(0, head), :],
                        acc_out.at[c].at[pl.ds(0, head), :])


def _sc_edge(h, asv, adv, eidx, n, e_tot, k):
    mesh = plsc.VectorSubcoreMesh(core_axis_name="c", subcore_axis_name="s")
    f32 = jnp.float32
    i32 = jnp.int32
    kern = pl.kernel(
        functools.partial(_sc_edge_body, n, e_tot, k),
        out_type=[jax.ShapeDtypeStruct((NC, n, D), f32),
                  jax.ShapeDtypeStruct((NC, NDEN), f32)],
        mesh=mesh,
        compiler_params=pltpu.CompilerParams(
            needs_layout_passes=False,
            disable_bounds_checks=True,
        ),
        scratch_types=[
            pltpu.VMEM((2, C), i32), pltpu.VMEM((2, C), i32),
            pltpu.VMEM((2, C), i32),
            pltpu.VMEM((C, D), f32), pltpu.VMEM((C, D), f32),
            pltpu.VMEM((C, D), f32),
            pltpu.VMEM((C,), f32), pltpu.VMEM((C,), f32),
            pltpu.VMEM((C,), f32), pltpu.VMEM((C,), f32),
            pltpu.VMEM((C,), f32), pltpu.VMEM((C,), f32),
            pltpu.VMEM((C,), i32), pltpu.VMEM((C,), i32),
            pltpu.VMEM((640,), f32),
            pltpu.VMEM_SHARED((n, D), f32),
            pltpu.VMEM_SHARED((NDEN,), f32),
        ] + [pltpu.SemaphoreType.DMA] * 14,
    )
    return kern(h, asv, adv, eidx)


def kernel(x, edge_index, W0, att_src0, att_dst0, b0, W1, att_src1, att_dst1,
           b1, W2, att_src2, att_dst2, b2, W_out, b_out):
    n = x.shape[0]
    src = edge_index[0]
    dst = edge_index[1]
    e_tot = edge_index.shape[1]

    per_worker = -(-e_tot // (NC * NS * C))
    k = -(-per_worker // 6) * 6
    e_pad = NC * NS * C * k
    fill = jnp.arange(e_pad - e_tot, dtype=src.dtype) % n
    srcp = jnp.concatenate([src, fill])
    dstp = jnp.concatenate([dst, fill])
    eidx = jnp.stack([srcp.reshape(-1, C), dstp.reshape(-1, C)], axis=1)

    r = 2000
    a0 = jnp.stack([att_src0, att_dst0], axis=1)
    a1 = jnp.stack([att_src1, att_dst1], axis=1)
    a2 = jnp.stack([att_src2, att_dst2], axis=1)

    h, av = _tc_first(x, W0, a0, n, r)
    acc, den = _sc_edge(h, av[:, 0], av[:, 1], eidx, n, e_tot, k)
    h2, av2 = _tc_mid(acc, den, h, av, b0, W1, a1, n, r)
    acc, den = _sc_edge(h2, av2[:, 0], av2[:, 1], eidx, n, e_tot, k)
    h3, av3 = _tc_mid(acc, den, h2, av2, b1, W2, a2, n, r)
    acc, den = _sc_edge(h3, av3[:, 0], av3[:, 1], eidx, n, e_tot, k)
    out = _tc_last(acc, den, h3, av3, b2, W_out, b_out, n, r)
    return out.reshape(n)
